# baseline (speedup 1.0000x reference)
import functools

import jax
import jax.numpy as jnp
from jax import lax
from jax.experimental import pallas as pl
from jax.experimental.pallas import tpu as pltpu
from jax.experimental.pallas import tpu_sc as plsc

N = 4096
S = 1024
LANES = 16
CHUNKS = S // LANES


def _perm_xor(v, m):
    lane = jnp.arange(LANES, dtype=jnp.int32)
    return v.at[lane ^ m].get(mode="promise_in_bounds", unique_indices=True)


def _combine(x, y, m):
    lane = jnp.arange(LANES, dtype=jnp.int32)
    take_x = (lane & m) == 0
    t1 = jnp.where(take_x, x, y)
    t2 = jnp.where(take_x, y, x)
    return t1 + _perm_xor(t2, m)


def _sums8(accs):
    c = [_combine(accs[2 * k], accs[2 * k + 1], 1) for k in range(4)]
    d = [_combine(c[2 * k], c[2 * k + 1], 2) for k in range(2)]
    e = _combine(d[0], d[1], 4)
    return e + _perm_xor(e, 8)


def _dot8(wt_vmem, vin_vmem, row_base):
    def chunk_body(c, accs):
        accs = list(accs)
        vc = vin_vmem[pl.ds(c * LANES, LANES)]
        for r in range(8):
            accs[r] = accs[r] + vc * wt_vmem[row_base + r, pl.ds(c * LANES, LANES)]
        return tuple(accs)

    accs0 = tuple(jnp.zeros((LANES,), jnp.float32) for _ in range(8))
    accs = lax.fori_loop(0, CHUNKS, chunk_body, accs0, unroll=8)
    return _sums8(accs)


def _dot_rows(wt_vmem, vin_vmem, vout_vmem, out_base):
    lane = jnp.arange(LANES, dtype=jnp.int32)

    def group_body(g, _):
        fa = _dot8(wt_vmem, vin_vmem, g * LANES)
        fb = _dot8(wt_vmem, vin_vmem, g * LANES + 8)
        vout_vmem[pl.ds(out_base + g * LANES, LANES)] = jnp.where(lane < 8, fa, fb)
        return 0

    lax.fori_loop(0, 2, group_body, 0)


def _bias_act(vout_vmem, b_vmem, nrows, b_base, apply_silu):
    for k in range(nrows // LANES):
        sl = pl.ds(k * LANES, LANES)
        a = vout_vmem[sl] + b_vmem[pl.ds(b_base + k * LANES, LANES)]
        if apply_silu:
            a = a / (1.0 + jnp.exp(-a))
        vout_vmem[sl] = a


def _mlp_body(pm_hbm, x_hbm, b_hbm, out_hbm,
              bufa, bufb, vin_vmem, vout_vmem, b_vmem, shared1, shared2,
              sema, semb):
    cid = lax.axis_index("c")
    sid = lax.axis_index("s")
    wid = cid * 16 + sid
    out0 = wid * 32

    r1 = S + sid * 64
    d0 = pltpu.async_copy(pm_hbm.at[pl.ds(r1, 32), pl.ds(0, S)], bufa, sema)
    d1 = pltpu.async_copy(pm_hbm.at[pl.ds(r1 + 32, 32), pl.ds(0, S)], bufb, semb)
    pltpu.sync_copy(x_hbm, vin_vmem)
    pltpu.sync_copy(b_hbm.at[pl.ds(sid * 64, 64)], b_vmem.at[pl.ds(0, 64)])
    pltpu.sync_copy(b_hbm.at[pl.ds(S + sid * 64, 64)], b_vmem.at[pl.ds(64, 64)])
    pltpu.sync_copy(b_hbm.at[pl.ds(2 * S + out0, 32)], b_vmem.at[pl.ds(128, 32)])

    r2 = 2 * S + sid * 64
    d0.wait()
    _dot_rows(bufa, vin_vmem, vout_vmem, 0)
    d2 = pltpu.async_copy(pm_hbm.at[pl.ds(r2, 32), pl.ds(S, S)], bufa, sema)
    d1.wait()
    _dot_rows(bufb, vin_vmem, vout_vmem, 32)
    d3 = pltpu.async_copy(pm_hbm.at[pl.ds(r2 + 32, 32), pl.ds(S, S)], bufb, semb)
    _bias_act(vout_vmem, b_vmem, 64, 0, apply_silu=True)
    pltpu.sync_copy(vout_vmem, shared1.at[pl.ds(sid * 64, 64)])
    plsc.subcore_barrier()
    pltpu.sync_copy(shared1, vin_vmem)

    d2.wait()
    _dot_rows(bufa, vin_vmem, vout_vmem, 0)
    d4 = pltpu.async_copy(pm_hbm.at[pl.ds(3 * S + out0, 32), pl.ds(2 * S, S)],
                          bufa, sema)
    d3.wait()
    _dot_rows(bufb, vin_vmem, vout_vmem, 32)
    _bias_act(vout_vmem, b_vmem, 64, 64, apply_silu=True)
    pltpu.sync_copy(vout_vmem, shared2.at[pl.ds(sid * 64, 64)])
    plsc.subcore_barrier()
    pltpu.sync_copy(shared2, vin_vmem)

    d4.wait()
    _dot_rows(bufa, vin_vmem, vout_vmem, 0)
    _bias_act(vout_vmem, b_vmem, 32, 128, apply_silu=False)
    pltpu.sync_copy(vout_vmem.at[pl.ds(0, 32)], out_hbm.at[pl.ds(out0, 32)])


def kernel(x, parameter_matrix):
    b_all = parameter_matrix[S:, N]

    mesh = plsc.VectorSubcoreMesh(core_axis_name="c", subcore_axis_name="s")
    k = functools.partial(
        pl.kernel,
        mesh=mesh,
        out_type=jax.ShapeDtypeStruct((S,), jnp.float32),
        compiler_params=pltpu.CompilerParams(use_tc_tiling_on_sc=True),
        scratch_types=[
            pltpu.VMEM((32, S), jnp.float32),
            pltpu.VMEM((32, S), jnp.float32),
            pltpu.VMEM((S,), jnp.float32),
            pltpu.VMEM((64,), jnp.float32),
            pltpu.VMEM((160,), jnp.float32),
            pltpu.VMEM_SHARED((S,), jnp.float32),
            pltpu.VMEM_SHARED((S,), jnp.float32),
            pltpu.SemaphoreType.DMA,
            pltpu.SemaphoreType.DMA,
        ],
    )(_mlp_body)
    return k(parameter_matrix, x, b_all)


# speedup vs baseline: 1.6018x; 1.6018x over previous
import functools

import jax
import jax.numpy as jnp
from jax import lax
from jax.experimental import pallas as pl
from jax.experimental.pallas import tpu as pltpu
from jax.experimental.pallas import tpu_sc as plsc

N = 4096
S = 1024
LANES = 16
CHUNKS = S // LANES


def _perm_xor(v, m):
    lane = jnp.arange(LANES, dtype=jnp.int32)
    return v.at[lane ^ m].get(mode="promise_in_bounds", unique_indices=True)


def _combine(x, y, m):
    lane = jnp.arange(LANES, dtype=jnp.int32)
    take_x = (lane & m) == 0
    t1 = jnp.where(take_x, x, y)
    t2 = jnp.where(take_x, y, x)
    return t1 + _perm_xor(t2, m)


def _sums8(accs):
    c = [_combine(accs[2 * k], accs[2 * k + 1], 1) for k in range(4)]
    d = [_combine(c[2 * k], c[2 * k + 1], 2) for k in range(2)]
    e = _combine(d[0], d[1], 4)
    return e + _perm_xor(e, 8)


def _dot8(wt_vmem, vin_vmem, row_base):
    def chunk_body(c, accs):
        accs = list(accs)
        vc = vin_vmem[pl.ds(c * LANES, LANES)]
        for r in range(8):
            accs[r] = accs[r] + vc * wt_vmem[row_base + r, pl.ds(c * LANES, LANES)]
        return tuple(accs)

    accs0 = tuple(jnp.zeros((LANES,), jnp.float32) for _ in range(8))
    accs = lax.fori_loop(0, CHUNKS, chunk_body, accs0, unroll=8)
    return _sums8(accs)


def _dot_rows(wt_vmem, vin_vmem, vout_vmem, out_base):
    lane = jnp.arange(LANES, dtype=jnp.int32)

    def group_body(g, _):
        fa = _dot8(wt_vmem, vin_vmem, g * LANES)
        fb = _dot8(wt_vmem, vin_vmem, g * LANES + 8)
        vout_vmem[pl.ds(out_base + g * LANES, LANES)] = jnp.where(lane < 8, fa, fb)
        return 0

    lax.fori_loop(0, 2, group_body, 0)


def _bias_act(vout_vmem, b_vmem, nrows, b_base, apply_silu):
    for k in range(nrows // LANES):
        sl = pl.ds(k * LANES, LANES)
        a = vout_vmem[sl] + b_vmem[pl.ds(b_base + k * LANES, LANES)]
        if apply_silu:
            a = a / (1.0 + jnp.exp(-a))
        vout_vmem[sl] = a


def _mlp_body(w1_hbm, w2_hbm, w3_hbm, x_hbm, b_hbm, out_hbm,
              bufa, bufb, vin_vmem, vout_vmem, b_vmem, shared1, shared2,
              sema, semb):
    cid = lax.axis_index("c")
    sid = lax.axis_index("s")
    wid = cid * 16 + sid
    out0 = wid * 32

    r1 = sid * 64
    d0 = pltpu.async_copy(w1_hbm.at[pl.ds(r1, 32), pl.ds(0, S)], bufa, sema)
    d1 = pltpu.async_copy(w1_hbm.at[pl.ds(r1 + 32, 32), pl.ds(0, S)], bufb, semb)
    pltpu.sync_copy(x_hbm, vin_vmem)
    pltpu.sync_copy(b_hbm.at[pl.ds(sid * 64, 64)], b_vmem.at[pl.ds(0, 64)])
    pltpu.sync_copy(b_hbm.at[pl.ds(S + sid * 64, 64)], b_vmem.at[pl.ds(64, 64)])
    pltpu.sync_copy(b_hbm.at[pl.ds(2 * S + out0, 32)], b_vmem.at[pl.ds(128, 32)])

    r2 = sid * 64
    d0.wait()
    _dot_rows(bufa, vin_vmem, vout_vmem, 0)
    d2 = pltpu.async_copy(w2_hbm.at[pl.ds(r2, 32), pl.ds(0, S)], bufa, sema)
    d1.wait()
    _dot_rows(bufb, vin_vmem, vout_vmem, 32)
    d3 = pltpu.async_copy(w2_hbm.at[pl.ds(r2 + 32, 32), pl.ds(0, S)], bufb, semb)
    _bias_act(vout_vmem, b_vmem, 64, 0, apply_silu=True)
    pltpu.sync_copy(vout_vmem, shared1.at[pl.ds(sid * 64, 64)])
    plsc.subcore_barrier()
    pltpu.sync_copy(shared1, vin_vmem)

    d2.wait()
    _dot_rows(bufa, vin_vmem, vout_vmem, 0)
    d4 = pltpu.async_copy(w3_hbm.at[pl.ds(out0, 32), pl.ds(0, S)], bufa, sema)
    d3.wait()
    _dot_rows(bufb, vin_vmem, vout_vmem, 32)
    _bias_act(vout_vmem, b_vmem, 64, 64, apply_silu=True)
    pltpu.sync_copy(vout_vmem, shared2.at[pl.ds(sid * 64, 64)])
    plsc.subcore_barrier()
    pltpu.sync_copy(shared2, vin_vmem)

    d4.wait()
    _dot_rows(bufa, vin_vmem, vout_vmem, 0)
    _bias_act(vout_vmem, b_vmem, 32, 128, apply_silu=False)
    pltpu.sync_copy(vout_vmem.at[pl.ds(0, 32)], out_hbm.at[pl.ds(out0, 32)])


def kernel(x, parameter_matrix):
    w1 = parameter_matrix[S:2 * S, 0:S]
    w2 = parameter_matrix[2 * S:3 * S, S:2 * S]
    w3 = parameter_matrix[3 * S:4 * S, 2 * S:3 * S]
    b_all = parameter_matrix[S:, N]

    mesh = plsc.VectorSubcoreMesh(core_axis_name="c", subcore_axis_name="s")
    k = functools.partial(
        pl.kernel,
        mesh=mesh,
        out_type=jax.ShapeDtypeStruct((S,), jnp.float32),
        compiler_params=pltpu.CompilerParams(use_tc_tiling_on_sc=True),
        scratch_types=[
            pltpu.VMEM((32, S), jnp.float32),
            pltpu.VMEM((32, S), jnp.float32),
            pltpu.VMEM((S,), jnp.float32),
            pltpu.VMEM((64,), jnp.float32),
            pltpu.VMEM((160,), jnp.float32),
            pltpu.VMEM_SHARED((S,), jnp.float32),
            pltpu.VMEM_SHARED((S,), jnp.float32),
            pltpu.SemaphoreType.DMA,
            pltpu.SemaphoreType.DMA,
        ],
    )(_mlp_body)
    return k(w1, w2, w3, x, b_all)
